# baseline (speedup 1.0000x reference)
import functools

import jax
import jax.numpy as jnp
from jax import lax
from jax.experimental import pallas as pl
from jax.experimental.pallas import tpu as pltpu
from jax.experimental.pallas import tpu_sc as plsc

VOCAB_ = 100000
EMB_ = 512
HID_ = 512
ENDS_ = (0, 20000, 60000, 100000)
ROWS_PER_BLOCK = 800
SCALE_ = float(HID_) ** 0.5


def _table_body(emb0, emb1, emb2, p0, p1, p2, out):
    g = pl.program_id(0)
    nb0 = (ENDS_[1] - ENDS_[0]) // ROWS_PER_BLOCK
    nb1 = (ENDS_[2] - ENDS_[1]) // ROWS_PER_BLOCK

    def proj(eref, pref):
        return lax.dot_general(
            eref[...], pref[...], (((1,), (1,)), ((), ())),
            preferred_element_type=jnp.float32,
        ) * SCALE_

    @pl.when(g < nb0)
    def _():
        out[...] = proj(emb0, p0)

    @pl.when((g >= nb0) & (g < nb0 + nb1))
    def _():
        out[...] = proj(emb1, p1)

    @pl.when(g >= nb0 + nb1)
    def _():
        out[...] = proj(emb2, p2)


def _build_table(emb_0, emb_1, emb_2, proj_0, proj_1, proj_2):
    r = ROWS_PER_BLOCK
    nb0 = (ENDS_[1] - ENDS_[0]) // r
    nb1 = (ENDS_[2] - ENDS_[1]) // r
    nb2 = (ENDS_[3] - ENDS_[2]) // r
    grid = nb0 + nb1 + nb2
    return pl.pallas_call(
        _table_body,
        grid=(grid,),
        in_specs=[
            pl.BlockSpec((r, EMB_), lambda g: (jnp.minimum(g, nb0 - 1), 0)),
            pl.BlockSpec((r, EMB_ // 2),
                         lambda g: (jnp.clip(g - nb0, 0, nb1 - 1), 0)),
            pl.BlockSpec((r, EMB_ // 4),
                         lambda g: (jnp.clip(g - nb0 - nb1, 0, nb2 - 1), 0)),
            pl.BlockSpec((HID_, EMB_), lambda g: (0, 0)),
            pl.BlockSpec((HID_, EMB_ // 2), lambda g: (0, 0)),
            pl.BlockSpec((HID_, EMB_ // 4), lambda g: (0, 0)),
        ],
        out_specs=pl.BlockSpec((r, HID_), lambda g: (g, 0)),
        out_shape=jax.ShapeDtypeStruct((VOCAB_, HID_), jnp.float32),
    )(emb_0, emb_1, emb_2, proj_0, proj_1, proj_2)


@functools.cache
def _make_gather(n_batch, seq):
    info = plsc.get_sparse_core_info()
    nc, ns = info.num_cores, info.num_subcores
    nw = nc * ns
    stride = (seq + 7) // 8 * 8
    assert n_batch % (2 * nw) == 0
    b_per_w = n_batch // nw
    mesh = plsc.VectorSubcoreMesh(core_axis_name="c", subcore_axis_name="s")

    @functools.partial(
        pl.kernel,
        mesh=mesh,
        out_type=jax.ShapeDtypeStruct((n_batch, seq, HID_), jnp.float32),
        compiler_params=pltpu.CompilerParams(use_tc_tiling_on_sc=False),
        scratch_types=[
            pltpu.VMEM((b_per_w * stride,), jnp.int32),
            pltpu.VMEM((seq, HID_), jnp.float32),
            pltpu.VMEM((seq, HID_), jnp.float32),
            pltpu.SemaphoreType.DMA,
            pltpu.SemaphoreType.DMA,
        ],
    )
    def gather(table_hbm, idx_hbm, out_hbm, idx_v, rows_a, rows_b, sem_a,
               sem_b):
        wid = lax.axis_index("s") * nc + lax.axis_index("c")
        b_base = wid * b_per_w
        pltpu.sync_copy(
            idx_hbm.at[pl.ds(b_base * stride, b_per_w * stride)], idx_v)

        def body(i, _):
            b2 = i * 2
            cp_a = pltpu.async_copy(
                table_hbm.at[idx_v.at[pl.ds(b2 * stride, seq)]],
                rows_a, sem_a)
            cp_b = pltpu.async_copy(
                table_hbm.at[idx_v.at[pl.ds((b2 + 1) * stride, seq)]],
                rows_b, sem_b)
            cp_a.wait()
            pltpu.sync_copy(rows_a, out_hbm.at[b_base + b2])
            cp_b.wait()
            pltpu.sync_copy(rows_b, out_hbm.at[b_base + b2 + 1])
            return ()

        lax.fori_loop(0, b_per_w // 2, body, (), unroll=False)

    return gather


def kernel(token_ids, emb_0, emb_1, emb_2, proj_0, proj_1, proj_2):
    table = _build_table(emb_0, emb_1, emb_2, proj_0, proj_1, proj_2)
    n_batch, seq = token_ids.shape
    stride = (seq + 7) // 8 * 8
    ids = jnp.pad(token_ids.astype(jnp.int32), ((0, 0), (0, stride - seq)))
    out = _make_gather(n_batch, seq)(table, ids.reshape(-1))
    return out


# speedup vs baseline: 1.0582x; 1.0582x over previous
import functools

import jax
import jax.numpy as jnp
from jax import lax
from jax.experimental import pallas as pl
from jax.experimental.pallas import tpu as pltpu
from jax.experimental.pallas import tpu_sc as plsc

VOCAB_ = 100000
EMB_ = 512
HID_ = 512
ENDS_ = (0, 20000, 60000, 100000)
ROWS_PER_BLOCK = 2000
NP_ = HID_ // 128
SCALE_ = float(HID_) ** 0.5


def _table_body(emb0, emb1, emb2, p0, p1, p2, out):
    g = pl.program_id(0)
    nb0 = (ENDS_[1] - ENDS_[0]) // ROWS_PER_BLOCK
    nb1 = (ENDS_[2] - ENDS_[1]) // ROWS_PER_BLOCK

    def proj(eref, pref):
        return lax.dot_general(
            eref[...], pref[...], (((1,), (1,)), ((), ())),
            preferred_element_type=jnp.float32,
        )[None] * SCALE_

    @pl.when(g < nb0)
    def _():
        out[...] = proj(emb0, p0)

    @pl.when((g >= nb0) & (g < nb0 + nb1))
    def _():
        out[...] = proj(emb1, p1)

    @pl.when(g >= nb0 + nb1)
    def _():
        out[...] = proj(emb2, p2)


def _build_table(emb_0, emb_1, emb_2, proj_0, proj_1, proj_2):
    r = ROWS_PER_BLOCK
    nb0 = (ENDS_[1] - ENDS_[0]) // r
    nb1 = (ENDS_[2] - ENDS_[1]) // r
    nb2 = (ENDS_[3] - ENDS_[2]) // r
    grid = (nb0 + nb1 + nb2, NP_)
    return pl.pallas_call(
        _table_body,
        grid=grid,
        in_specs=[
            pl.BlockSpec((r, EMB_), lambda g, k: (jnp.minimum(g, nb0 - 1), 0)),
            pl.BlockSpec((r, EMB_ // 2),
                         lambda g, k: (jnp.clip(g - nb0, 0, nb1 - 1), 0)),
            pl.BlockSpec((r, EMB_ // 4),
                         lambda g, k: (jnp.clip(g - nb0 - nb1, 0, nb2 - 1), 0)),
            pl.BlockSpec((128, EMB_), lambda g, k: (k, 0)),
            pl.BlockSpec((128, EMB_ // 2), lambda g, k: (k, 0)),
            pl.BlockSpec((128, EMB_ // 4), lambda g, k: (k, 0)),
        ],
        out_specs=pl.BlockSpec((1, r, 128), lambda g, k: (k, g, 0)),
        out_shape=jax.ShapeDtypeStruct((NP_, VOCAB_, 128), jnp.float32),
    )(emb_0, emb_1, emb_2, proj_0, proj_1, proj_2)


@functools.cache
def _make_gather(n_batch, seq):
    info = plsc.get_sparse_core_info()
    nc, ns = info.num_cores, info.num_subcores
    nw = nc * ns
    rps = seq * NP_
    bpc = 2
    crows = bpc * rps
    assert n_batch % (2 * nw) == 0 and rps % 8 == 0
    b_per_w = n_batch // nw
    n_chunk = b_per_w // bpc
    splits = []
    off = 0
    while off < crows:
        n = min(128, crows - off)
        splits.append((off, n))
        off += n
    mesh = plsc.VectorSubcoreMesh(core_axis_name="c", subcore_axis_name="s")

    @functools.partial(
        pl.kernel,
        mesh=mesh,
        out_type=jax.ShapeDtypeStruct((n_batch * seq * NP_, 128), jnp.float32),
        compiler_params=pltpu.CompilerParams(use_tc_tiling_on_sc=False),
        scratch_types=[
            pltpu.VMEM((b_per_w * rps,), jnp.int32),
            pltpu.VMEM((crows, 128), jnp.float32),
            pltpu.VMEM((crows, 128), jnp.float32),
            pltpu.SemaphoreType.DMA,
            pltpu.SemaphoreType.DMA,
        ],
    )
    def gather(table_hbm, idx_hbm, out_lin, idx_v, rows_a, rows_b, sem_a,
               sem_b):
        wid = lax.axis_index("s") * nc + lax.axis_index("c")
        base = wid * b_per_w * rps
        pltpu.sync_copy(idx_hbm.at[pl.ds(base, b_per_w * rps)], idx_v)

        def fire(c, buf, sem):
            return [
                pltpu.async_copy(
                    table_hbm.at[idx_v.at[pl.ds(c * crows + o, n)]],
                    buf.at[pl.ds(o, n)], sem)
                for o, n in splits
            ]

        def body(i, _):
            c2 = i * 2
            cps_a = fire(c2, rows_a, sem_a)
            cps_b = fire(c2 + 1, rows_b, sem_b)
            for cp in cps_a:
                cp.wait()
            pltpu.sync_copy(rows_a,
                            out_lin.at[pl.ds(base + c2 * crows, crows)])
            for cp in cps_b:
                cp.wait()
            pltpu.sync_copy(rows_b,
                            out_lin.at[pl.ds(base + (c2 + 1) * crows, crows)])
            return ()

        lax.fori_loop(0, n_chunk // 2, body, (), unroll=False)

    return gather


def kernel(token_ids, emb_0, emb_1, emb_2, proj_0, proj_1, proj_2):
    table = _build_table(emb_0, emb_1, emb_2, proj_0, proj_1, proj_2)
    n_batch, seq = token_ids.shape
    ids = token_ids.astype(jnp.int32)
    idx4 = (ids[:, :, None]
            + jnp.arange(NP_, dtype=jnp.int32) * VOCAB_).reshape(-1)
    out = _make_gather(n_batch, seq)(
        table.reshape(NP_ * VOCAB_, 128), idx4)
    return out.reshape(n_batch, seq, HID_)
